# Initial kernel scaffold; baseline (speedup 1.0000x reference)
import functools

import jax
import jax.numpy as jnp
from jax import lax
from jax.experimental import pallas as pl
from jax.experimental.pallas import tpu as pltpu
from jax.experimental.pallas import tpu_sc as plsc

N_NODES = 10000
D_FEAT = 128
N_EDGES = 320000

NC = 2
NS = 16
NW = NC * NS
EW = N_EDGES // NW
K = 80
NCHUNK = EW // K
RPT = N_NODES // NS
RZ = 125
NB = D_FEAT // 16


def _logmap0_body(x_ref, o_ref):
    x = x_ref[...]
    sq = jnp.sum(x * x, axis=-1, keepdims=True)
    norm = jnp.maximum(jnp.sqrt(sq), 1e-15)
    scn = jnp.minimum(norm, 1.0 - 1e-5)
    atanh = 0.5 * jnp.log((1.0 + scn) / (1.0 - scn))
    o_ref[...] = atanh * x / norm


def _logmap0(x):
    return pl.pallas_call(
        _logmap0_body,
        out_shape=jax.ShapeDtypeStruct((N_NODES, D_FEAT), jnp.float32),
    )(x)


def _add2_body(p_ref, o_ref):
    o_ref[...] = p_ref[0] + p_ref[1]


def _add2(p):
    return pl.pallas_call(
        _add2_body,
        out_shape=jax.ShapeDtypeStruct((N_NODES, D_FEAT), jnp.float32),
    )(p)


def _add3_body(y_ref, p_ref, o_ref):
    o_ref[...] = y_ref[...] + p_ref[0] + p_ref[1]


def _add3(y, p):
    return pl.pallas_call(
        _add3_body,
        out_shape=jax.ShapeDtypeStruct((N_NODES, D_FEAT), jnp.float32),
    )(y, p)


def _spmm_body(h_hbm, col_hbm, row_hbm, w_hbm, out_hbm,
               colv, rowv, wv, rows_v, zbuf, acc, sem):
    c = lax.axis_index("c")
    s = lax.axis_index("s")
    wid = s * NC + c

    zero16 = jnp.zeros((16,), jnp.float32)

    def zrow(r, carry):
        for b in range(NB):
            zbuf[r, pl.ds(b * 16, 16)] = zero16
        return carry

    lax.fori_loop(0, RZ, zrow, 0)
    for i in range(RPT // RZ):
        pltpu.sync_copy(zbuf, acc.at[pl.ds(s * RPT + i * RZ, RZ)])
    plsc.subcore_barrier()

    e0 = wid * EW

    def chunk(g, carry):
        e = e0 + g * K
        pltpu.sync_copy(col_hbm.at[pl.ds(e, K)], colv)
        pltpu.sync_copy(row_hbm.at[pl.ds(e, K)], rowv)
        pltpu.sync_copy(w_hbm.at[pl.ds(e, K)], wv)
        pltpu.async_copy(h_hbm.at[colv], rows_v, sem).wait()

        def srow(j, inner):
            wj = plsc.load_gather(wv, [jnp.zeros((16,), jnp.int32) + j])
            for b in range(NB):
                rows_v[j, pl.ds(b * 16, 16)] = rows_v[j, pl.ds(b * 16, 16)] * wj
            return inner

        lax.fori_loop(0, K, srow, 0, unroll=2)
        pltpu.sync_copy(rows_v, acc.at[rowv], add=True)
        return carry

    lax.fori_loop(0, NCHUNK, chunk, 0)
    plsc.subcore_barrier()

    for i in range(RPT // RZ):
        off = s * RPT + i * RZ
        pltpu.sync_copy(acc.at[pl.ds(off, RZ)], out_hbm.at[c, pl.ds(off, RZ)])


def _spmm_sc(h, col, row, w):
    mesh = plsc.VectorSubcoreMesh(core_axis_name="c", subcore_axis_name="s")
    f = pl.kernel(
        _spmm_body,
        out_type=jax.ShapeDtypeStruct((NC, N_NODES, D_FEAT), jnp.float32),
        mesh=mesh,
        scratch_types=[
            pltpu.VMEM((K,), jnp.int32),
            pltpu.VMEM((K,), jnp.int32),
            pltpu.VMEM((K,), jnp.float32),
            pltpu.VMEM((K, D_FEAT), jnp.float32),
            pltpu.VMEM((RZ, D_FEAT), jnp.float32),
            pltpu.VMEM_SHARED((N_NODES, D_FEAT), jnp.float32),
            pltpu.SemaphoreType.DMA,
        ],
    )
    return f(h, col, row, w)


def kernel(x, edge_index, edge_weight):
    ei = edge_index.astype(jnp.int32)
    row = ei[0]
    col = ei[1]
    w = edge_weight.astype(jnp.float32)

    h = _logmap0(x.astype(jnp.float32))
    p1 = _spmm_sc(h, col, row, w)
    y1 = _add2(p1)
    p2 = _spmm_sc(y1, col, row, w)
    return _add3(y1, p2)


# speedup vs baseline: 4.0579x; 4.0579x over previous
import functools

import jax
import jax.numpy as jnp
from jax import lax
from jax.experimental import pallas as pl
from jax.experimental.pallas import tpu as pltpu
from jax.experimental.pallas import tpu_sc as plsc

N_NODES = 10000
D_FEAT = 128
N_EDGES = 320000

NC = 2
NS = 16
NW = NC * NS
EW = N_EDGES // NW
K = 80
NCHUNK = EW // K
N_PAD = 10240
RPT = N_PAD // NS
RZ = 80
NB = D_FEAT // 16


def _logmap0_body(x_ref, o_ref):
    x = x_ref[...]
    sq = jnp.sum(x * x, axis=-1, keepdims=True)
    norm = jnp.maximum(jnp.sqrt(sq), 1e-15)
    scn = jnp.minimum(norm, 1.0 - 1e-5)
    atanh = 0.5 * jnp.log((1.0 + scn) / (1.0 - scn))
    o_ref[...] = atanh * x / norm


def _logmap0(x):
    return pl.pallas_call(
        _logmap0_body,
        out_shape=jax.ShapeDtypeStruct((N_NODES, D_FEAT), jnp.float32),
    )(x)


def _add2_body(p_ref, o_ref):
    o_ref[...] = p_ref[0] + p_ref[1]


def _add2(p):
    return pl.pallas_call(
        _add2_body,
        out_shape=jax.ShapeDtypeStruct((N_NODES, D_FEAT), jnp.float32),
    )(p)


def _add3_body(y_ref, p_ref, o_ref):
    o_ref[...] = y_ref[...] + p_ref[0] + p_ref[1]


def _add3(y, p):
    return pl.pallas_call(
        _add3_body,
        out_shape=jax.ShapeDtypeStruct((N_NODES, D_FEAT), jnp.float32),
    )(y, p)


def _spmm_body(h_hbm, col_hbm, row_hbm, w_hbm, out_hbm,
               colv, rowv, wv, rows_v, zbuf, acc, sem):
    c = lax.axis_index("c")
    s = lax.axis_index("s")
    wid = s * NC + c

    zero16 = jnp.zeros((16,), jnp.float32)

    def zrow(r, carry):
        for b in range(NB):
            zbuf[r, pl.ds(b * 16, 16)] = zero16
        return carry

    lax.fori_loop(0, RZ, zrow, 0)
    for i in range(RPT // RZ):
        pltpu.sync_copy(zbuf, acc.at[pl.ds(s * RPT + i * RZ, RZ)])
    plsc.subcore_barrier()

    e0 = wid * EW

    def chunk(g, carry):
        e = e0 + g * K
        pltpu.sync_copy(col_hbm.at[pl.ds(e, K)], colv)
        pltpu.sync_copy(row_hbm.at[pl.ds(e, K)], rowv)
        pltpu.sync_copy(w_hbm.at[pl.ds(e, K)], wv)
        pltpu.async_copy(h_hbm.at[colv], rows_v, sem).wait()

        def grp(g16, inner):
            base = g16 * 16
            w16 = wv[pl.ds(base, 16)]
            dn = lax.GatherDimensionNumbers(
                offset_dims=(), collapsed_slice_dims=(0,),
                start_index_map=(0,))
            for j in range(16):
                wj = lax.gather(
                    w16, jnp.full((16, 1), j, jnp.int32), dn, (1,),
                    mode=lax.GatherScatterMode.PROMISE_IN_BOUNDS)
                for b in range(NB):
                    r = base + j
                    rows_v[r, pl.ds(b * 16, 16)] = (
                        rows_v[r, pl.ds(b * 16, 16)] * wj)
            return inner

        lax.fori_loop(0, K // 16, grp, 0)
        pltpu.sync_copy(rows_v, acc.at[rowv], add=True)
        return carry

    lax.fori_loop(0, NCHUNK, chunk, 0)
    plsc.subcore_barrier()

    for i in range(RPT // RZ):
        off = s * RPT + i * RZ

        @pl.when(off < N_NODES)
        def _():
            pltpu.sync_copy(acc.at[pl.ds(off, RZ)], out_hbm.at[c, pl.ds(off, RZ)])


def _spmm_sc(h, col, row, w):
    mesh = plsc.VectorSubcoreMesh(core_axis_name="c", subcore_axis_name="s")
    f = pl.kernel(
        _spmm_body,
        out_type=jax.ShapeDtypeStruct((NC, N_NODES, D_FEAT), jnp.float32),
        mesh=mesh,
        scratch_types=[
            pltpu.VMEM((K,), jnp.int32),
            pltpu.VMEM((K,), jnp.int32),
            pltpu.VMEM((K,), jnp.float32),
            pltpu.VMEM((K, D_FEAT), jnp.float32),
            pltpu.VMEM((RZ, D_FEAT), jnp.float32),
            pltpu.VMEM_SHARED((N_PAD, D_FEAT), jnp.float32),
            pltpu.SemaphoreType.DMA,
        ],
    )
    return f(h, col, row, w)


def kernel(x, edge_index, edge_weight):
    ei = edge_index.astype(jnp.int32)
    row = ei[0]
    col = ei[1]
    w = edge_weight.astype(jnp.float32)

    h = _logmap0(x.astype(jnp.float32))
    p1 = _spmm_sc(h, col, row, w)
    y1 = _add2(p1)
    p2 = _spmm_sc(y1, col, row, w)
    return _add3(y1, p2)


# speedup vs baseline: 10.3463x; 2.5497x over previous
import functools

import jax
import jax.numpy as jnp
from jax import lax
from jax.experimental import pallas as pl
from jax.experimental.pallas import tpu as pltpu
from jax.experimental.pallas import tpu_sc as plsc

N_NODES = 10000
D_FEAT = 128
N_EDGES = 320000

NC = 2
NS = 16
NW = NC * NS
EW = N_EDGES // NW
K = 80
NCHUNK = EW // K
NBUF = 4
N_PAD = 10240
RPT = N_PAD // NS
RZ = 80
NB = D_FEAT // 16


def _logmap0_body(x_ref, o_ref):
    x = x_ref[...]
    sq = jnp.sum(x * x, axis=-1, keepdims=True)
    norm = jnp.maximum(jnp.sqrt(sq), 1e-15)
    scn = jnp.minimum(norm, 1.0 - 1e-5)
    atanh = 0.5 * jnp.log((1.0 + scn) / (1.0 - scn))
    o_ref[...] = atanh * x / norm


def _logmap0(x):
    return pl.pallas_call(
        _logmap0_body,
        out_shape=jax.ShapeDtypeStruct((N_NODES, D_FEAT), jnp.float32),
    )(x)


def _add2_body(p_ref, o_ref):
    o_ref[...] = p_ref[0] + p_ref[1]


def _add2(p):
    return pl.pallas_call(
        _add2_body,
        out_shape=jax.ShapeDtypeStruct((N_NODES, D_FEAT), jnp.float32),
    )(p)


def _add3_body(y_ref, p_ref, o_ref):
    o_ref[...] = y_ref[...] + p_ref[0] + p_ref[1]


def _add3(y, p):
    return pl.pallas_call(
        _add3_body,
        out_shape=jax.ShapeDtypeStruct((N_NODES, D_FEAT), jnp.float32),
    )(y, p)


def _spmm_body(h_hbm, col_hbm, row_hbm, w_hbm, out_hbm,
               cb0, cb1, cb2, cb3, rwb0, rwb1, rwb2, rwb3,
               wb0, wb1, wb2, wb3, rb0, rb1, acc,
               i0, i1, i2, i3, g0, g1, s0, s1):
    c = lax.axis_index("c")
    s = lax.axis_index("s")
    wid = s * NC + c
    cb = (cb0, cb1, cb2, cb3)
    rwb = (rwb0, rwb1, rwb2, rwb3)
    wb = (wb0, wb1, wb2, wb3)
    isem = (i0, i1, i2, i3)
    rbufs = (rb0, rb1)
    gsem = (g0, g1)
    ssem = (s0, s1)
    e0 = wid * EW

    def idx_start(gb, slot):
        e = e0 + gb * K
        pltpu.async_copy(col_hbm.at[pl.ds(e, K)], cb[slot], isem[slot])
        pltpu.async_copy(row_hbm.at[pl.ds(e, K)], rwb[slot], isem[slot])
        pltpu.async_copy(w_hbm.at[pl.ds(e, K)], wb[slot], isem[slot])

    def idx_wait(gb, slot):
        e = e0 + gb * K
        pltpu.make_async_copy(col_hbm.at[pl.ds(e, K)], cb[slot], isem[slot]).wait()
        pltpu.make_async_copy(row_hbm.at[pl.ds(e, K)], rwb[slot], isem[slot]).wait()
        pltpu.make_async_copy(w_hbm.at[pl.ds(e, K)], wb[slot], isem[slot]).wait()

    zero16 = jnp.zeros((16,), jnp.float32)

    def zrow(r, carry):
        for b in range(NB):
            rb0[r, pl.ds(b * 16, 16)] = zero16
        return carry

    lax.fori_loop(0, K, zrow, 0)
    for i in range(RPT // RZ):
        pltpu.sync_copy(rb0, acc.at[pl.ds(s * RPT + i * RZ, RZ)])
    plsc.subcore_barrier()

    def scale(buf, wslot):
        def grp(g16, inner):
            base = g16 * 16
            w16 = wb[wslot][pl.ds(base, 16)]
            dn = lax.GatherDimensionNumbers(
                offset_dims=(), collapsed_slice_dims=(0,),
                start_index_map=(0,))
            for j in range(16):
                wj = lax.gather(
                    w16, jnp.full((16, 1), j, jnp.int32), dn, (1,),
                    mode=lax.GatherScatterMode.PROMISE_IN_BOUNDS)
                for b in range(NB):
                    r = base + j
                    buf[r, pl.ds(b * 16, 16)] = buf[r, pl.ds(b * 16, 16)] * wj
            return inner

        lax.fori_loop(0, K // 16, grp, 0)

    idx_start(0, 0)
    idx_start(1, 1)
    idx_wait(0, 0)
    pltpu.async_copy(h_hbm.at[cb0], rb0, g0)

    def quad(q, carry):
        for b in range(4):
            gb = q * 4 + b
            rs = b % 2
            ns = (b + 1) % 2
            is2 = (b + 2) % 4
            is1 = (b + 1) % 4
            ip1 = (b + 3) % 4

            @pl.when(gb < NCHUNK)
            def _():
                @pl.when(gb >= 1)
                def _():
                    pltpu.make_async_copy(
                        rbufs[ns], acc.at[rwb[ip1]], ssem[ns]).wait()

                @pl.when(gb + 1 < NCHUNK)
                def _():
                    idx_wait(gb + 1, is1)
                    pltpu.async_copy(h_hbm.at[cb[is1]], rbufs[ns], gsem[ns])

                pltpu.make_async_copy(
                    h_hbm.at[cb[b % 4]], rbufs[rs], gsem[rs]).wait()
                scale(rbufs[rs], b % 4)
                pltpu.async_copy(
                    rbufs[rs], acc.at[rwb[b % 4]], ssem[rs], add=True)

                @pl.when(gb + 2 < NCHUNK)
                def _():
                    idx_start(gb + 2, is2)

        return carry

    lax.fori_loop(0, (NCHUNK + 3) // 4, quad, 0)

    last = NCHUNK - 1
    pltpu.make_async_copy(
        rbufs[last % 2], acc.at[rwb[last % 4]], ssem[last % 2]).wait()

    plsc.subcore_barrier()

    for i in range(RPT // RZ):
        off = s * RPT + i * RZ

        @pl.when(off < N_NODES)
        def _():
            pltpu.sync_copy(acc.at[pl.ds(off, RZ)], out_hbm.at[c, pl.ds(off, RZ)])


def _spmm_sc(h, col, row, w):
    mesh = plsc.VectorSubcoreMesh(core_axis_name="c", subcore_axis_name="s")
    f = pl.kernel(
        _spmm_body,
        out_type=jax.ShapeDtypeStruct((NC, N_NODES, D_FEAT), jnp.float32),
        mesh=mesh,
        scratch_types=(
            [pltpu.VMEM((K,), jnp.int32) for _ in range(4)]
            + [pltpu.VMEM((K,), jnp.int32) for _ in range(4)]
            + [pltpu.VMEM((K,), jnp.float32) for _ in range(4)]
            + [pltpu.VMEM((K, D_FEAT), jnp.float32) for _ in range(2)]
            + [pltpu.VMEM_SHARED((N_PAD, D_FEAT), jnp.float32)]
            + [pltpu.SemaphoreType.DMA for _ in range(8)]
        ),
    )
    return f(h, col, row, w)


def kernel(x, edge_index, edge_weight):
    ei = edge_index.astype(jnp.int32)
    row = ei[0]
    col = ei[1]
    w = edge_weight.astype(jnp.float32)

    h = _logmap0(x.astype(jnp.float32))
    p1 = _spmm_sc(h, col, row, w)
    y1 = _add2(p1)
    p2 = _spmm_sc(y1, col, row, w)
    return _add3(y1, p2)
